# baseline (speedup 1.0000x reference)
import jax
import jax.numpy as jnp
from jax.experimental import pallas as pl
from jax.experimental.pallas import tpu as pltpu

_TM = 256


def _left_kernel(x_ref, w_ref, wc_ref, o_ref, wbc_ref):
    wbc_ref[...] = wc_ref[...].astype(jnp.bfloat16)
    xb = x_ref[...].astype(jnp.bfloat16)
    wb = w_ref[...].astype(jnp.bfloat16)
    o_ref[...] = jnp.dot(xb, wb, preferred_element_type=jnp.float32)


def _right_kernel(x_ref, wb_ref, prev_ref, o_ref):
    del prev_ref
    xb = x_ref[...].astype(jnp.bfloat16)
    o_ref[...] = jnp.dot(xb, wb_ref[...], preferred_element_type=jnp.float32)


def kernel(x, weight_t_padded):
    M, K = x.shape
    Kp, N = weight_t_padded.shape
    tn = N // 2
    nsteps = M // _TM
    tc = tn // nsteps
    assert Kp == K and tc % 128 == 0, (M, K, Kp, N, tc)

    half0, wb1 = pl.pallas_call(
        _left_kernel,
        out_shape=[
            jax.ShapeDtypeStruct((M, N), jnp.float32),
            jax.ShapeDtypeStruct((K, tn), jnp.bfloat16),
        ],
        grid_spec=pltpu.PrefetchScalarGridSpec(
            num_scalar_prefetch=0,
            grid=(nsteps,),
            in_specs=[
                pl.BlockSpec((_TM, K), lambda i: (i, 0)),
                pl.BlockSpec((K, tn), lambda i: (0, 0)),
                pl.BlockSpec((K, tc), lambda i: (0, (tn // tc) + i)),
            ],
            out_specs=[
                pl.BlockSpec((_TM, tn), lambda i: (i, 0)),
                pl.BlockSpec((K, tc), lambda i: (0, i)),
            ],
        ),
        compiler_params=pltpu.CompilerParams(
            dimension_semantics=("arbitrary",),
            vmem_limit_bytes=64 * 1024 * 1024,
        ),
        cost_estimate=pl.CostEstimate(
            flops=2 * M * K * tn,
            transcendentals=0,
            bytes_accessed=M * K * 4 + K * N * 4 + M * tn * 4 + K * tn * 2,
        ),
    )(x, weight_t_padded, weight_t_padded)

    return pl.pallas_call(
        _right_kernel,
        out_shape=jax.ShapeDtypeStruct((M, N), jnp.float32),
        grid_spec=pltpu.PrefetchScalarGridSpec(
            num_scalar_prefetch=0,
            grid=(2, nsteps),
            in_specs=[
                pl.BlockSpec((_TM, K), lambda j, i: (i, 0)),
                pl.BlockSpec((K, tn // 2), lambda j, i: (0, j)),
                pl.BlockSpec(memory_space=pl.ANY),
            ],
            out_specs=pl.BlockSpec((_TM, tn // 2), lambda j, i: (i, 2 + j)),
        ),
        input_output_aliases={2: 0},
        compiler_params=pltpu.CompilerParams(
            dimension_semantics=("arbitrary", "arbitrary"),
            vmem_limit_bytes=64 * 1024 * 1024,
        ),
        cost_estimate=pl.CostEstimate(
            flops=2 * M * K * tn,
            transcendentals=0,
            bytes_accessed=M * K * 4 + K * tn * 2 + M * tn * 4,
        ),
    )(x, wb1, half0)


# speedup vs baseline: 1.0467x; 1.0467x over previous
import jax
import jax.numpy as jnp
from jax.experimental import pallas as pl
from jax.experimental.pallas import tpu as pltpu

_TM = 256


def _left_kernel(x_ref, w_ref, wc_ref, o_ref, wbc_ref):
    wbc_ref[...] = wc_ref[...].astype(jnp.bfloat16)
    xb = x_ref[...].astype(jnp.bfloat16)
    wb = w_ref[...].astype(jnp.bfloat16)
    o_ref[...] = jnp.dot(xb, wb, preferred_element_type=jnp.float32)


def _right_kernel(x_ref, wb_ref, prev_ref, o_ref):
    del prev_ref
    xb = x_ref[...].astype(jnp.bfloat16)
    o_ref[...] = jnp.dot(xb, wb_ref[...], preferred_element_type=jnp.float32)


def kernel(x, weight_t_padded):
    M, K = x.shape
    Kp, N = weight_t_padded.shape
    tn = N // 2
    nsteps = M // _TM
    tc = tn // nsteps
    assert Kp == K and tc % 128 == 0, (M, K, Kp, N, tc)

    half0, wb1 = pl.pallas_call(
        _left_kernel,
        out_shape=[
            jax.ShapeDtypeStruct((M, N), jnp.float32),
            jax.ShapeDtypeStruct((K, tn), jnp.bfloat16),
        ],
        grid_spec=pltpu.PrefetchScalarGridSpec(
            num_scalar_prefetch=0,
            grid=(nsteps,),
            in_specs=[
                pl.BlockSpec((_TM, K), lambda i: (i, 0)),
                pl.BlockSpec((K, tn), lambda i: (0, 0)),
                pl.BlockSpec((K, tc), lambda i: (0, (tn // tc) + i)),
            ],
            out_specs=[
                pl.BlockSpec((_TM, tn), lambda i: (i, 0)),
                pl.BlockSpec((K, tc), lambda i: (0, i)),
            ],
        ),
        compiler_params=pltpu.CompilerParams(
            dimension_semantics=("arbitrary",),
            vmem_limit_bytes=64 * 1024 * 1024,
        ),
        cost_estimate=pl.CostEstimate(
            flops=2 * M * K * tn,
            transcendentals=0,
            bytes_accessed=M * K * 4 + K * N * 4 + M * tn * 4 + K * tn * 2,
        ),
    )(x, weight_t_padded, weight_t_padded)

    return pl.pallas_call(
        _right_kernel,
        out_shape=jax.ShapeDtypeStruct((M, N), jnp.float32),
        grid_spec=pltpu.PrefetchScalarGridSpec(
            num_scalar_prefetch=0,
            grid=(nsteps,),
            in_specs=[
                pl.BlockSpec((_TM, K), lambda i: (i, 0)),
                pl.BlockSpec((K, tn), lambda i: (0, 0)),
                pl.BlockSpec(memory_space=pl.ANY),
            ],
            out_specs=pl.BlockSpec((_TM, tn), lambda i: (i, 1)),
        ),
        input_output_aliases={2: 0},
        compiler_params=pltpu.CompilerParams(
            dimension_semantics=("arbitrary",),
            vmem_limit_bytes=64 * 1024 * 1024,
        ),
        cost_estimate=pl.CostEstimate(
            flops=2 * M * K * tn,
            transcendentals=0,
            bytes_accessed=M * K * 4 + K * tn * 2 + M * tn * 4,
        ),
    )(x, wb1, half0)
